# baseline (speedup 1.0000x reference)
import functools

import jax
import jax.numpy as jnp
from jax import lax
from jax.experimental import pallas as pl
from jax.experimental.pallas import tpu as pltpu
from jax.experimental.pallas import tpu_sc as plsc

N = 10000
E = 320000
HIDDEN = 128
HEADS = 8
HD = HIDDEN // HEADS
EPS = 1e-5

NC = 2
NS = 16
L = 16

HEADS_C = HEADS // NC
QW = HEADS_C * HD
KVW = 2 * QW
ACC_W = 80

K = 128
NCHUNKS = E // K
CH_PER_T = -(-NCHUNKS // NS)


def _pre_body(x_ref, wq_ref, wkv_ref, q_ref, kv_ref):
    xb = x_ref[...]
    q_ref[0] = jnp.dot(xb, wq_ref[0], preferred_element_type=jnp.float32,
                       precision=lax.Precision.HIGHEST)
    kv_ref[0] = jnp.dot(xb, wkv_ref[0], preferred_element_type=jnp.float32,
                        precision=lax.Precision.HIGHEST)


def _tc_pre(x, Wq2, Wkv2):
    nb = 5
    rb = N // nb
    return pl.pallas_call(
        _pre_body,
        grid=(NC, nb),
        in_specs=[
            pl.BlockSpec((rb, HIDDEN), lambda c, j: (j, 0)),
            pl.BlockSpec((1, HIDDEN, QW), lambda c, j: (c, 0, 0)),
            pl.BlockSpec((1, HIDDEN, KVW), lambda c, j: (c, 0, 0)),
        ],
        out_specs=[
            pl.BlockSpec((1, rb, QW), lambda c, j: (c, j, 0)),
            pl.BlockSpec((1, rb, KVW), lambda c, j: (c, j, 0)),
        ],
        out_shape=[
            jax.ShapeDtypeStruct((NC, N, QW), jnp.float32),
            jax.ShapeDtypeStruct((NC, N, KVW), jnp.float32),
        ],
    )(x, Wq2, Wkv2)


def _compute_upd(est, q_rows, kv_rows, upd):
    lanes = lax.iota(jnp.int32, L)

    @plsc.parallel_loop(0, K // L, 1, unroll=4)
    def group_body(g):
        e0 = g * L
        for h in range(HEADS_C):
            sv = jnp.zeros((L,), jnp.float32)
            for i in range(L):
                e = e0 + i
                qv = q_rows[e, pl.ds(h * HD, HD)]
                kk = kv_rows[e, pl.ds(h * HD, HD)]
                sv = jnp.where(lanes == i, jnp.sum(qv * kk), sv)
            est[h, pl.ds(e0, L)] = jnp.exp(jnp.minimum(sv * 0.25, 60.0))
        esv = [est[h, pl.ds(e0, L)] for h in range(HEADS_C)]
        for i in range(L):
            e = e0 + i
            den = jnp.zeros((L,), jnp.float32)
            for h in range(HEADS_C):
                sb = esv[h].at[jnp.full((L,), i, jnp.int32)].get(
                    mode="promise_in_bounds")
                vv = kv_rows[e, pl.ds(QW + h * HD, HD)]
                upd[e, pl.ds(h * HD, HD)] = vv * sb
                den = jnp.where(lanes == h, sb, den)
            upd[e, pl.ds(QW, L)] = den


NT = 2 * (-(-(-(-NCHUNKS // NS)) // 2))


def _edge_body(q_hbm, kv_hbm, edges2_hbm, zeros_hbm, out_hbm,
               idx_a, idx_b, sidx_a, sidx_b, q_a, q_b, kv_a, kv_b,
               upd_a, upd_b, est, acc, gsem_a, gsem_b, ssem_a, ssem_b):
    c = lax.axis_index("c")
    s = lax.axis_index("s")

    @pl.when(s == 0)
    def _():
        pltpu.sync_copy(zeros_hbm, acc)

    plsc.subcore_barrier()

    def chunk_of(t):
        return s + t * NS

    def issue(t, idx2, qb, kvb, gsem):
        pltpu.sync_copy(edges2_hbm.at[chunk_of(t)], idx2)
        pltpu.async_copy(kv_hbm.at[c].at[idx2.at[0]], kvb, gsem)
        pltpu.async_copy(q_hbm.at[c].at[idx2.at[1]], qb, gsem)

    def consume(t, tt, idx2, sidx, qb, kvb, upd, gsem, ssem):
        pltpu.make_async_copy(kv_hbm.at[c].at[idx2.at[0]], kvb, gsem).wait()
        pltpu.make_async_copy(q_hbm.at[c].at[idx2.at[1]], qb, gsem).wait()

        @pl.when(tt > 0)
        def _():
            pltpu.make_async_copy(upd, acc.at[sidx.at[0]], ssem).wait()

        for i in range(K // L):
            sidx[0, pl.ds(i * L, L)] = idx2[1, pl.ds(i * L, L)]
        _compute_upd(est, qb, kvb, upd)
        pltpu.async_copy(upd, acc.at[sidx.at[0]], ssem, add=True)

    @pl.when(chunk_of(0) < NCHUNKS)
    def _():
        issue(0, idx_a, q_a, kv_a, gsem_a)

    def pair_body(tt, carry):
        t0 = 2 * tt
        t1 = 2 * tt + 1

        @pl.when(chunk_of(t1) < NCHUNKS)
        def _():
            issue(t1, idx_b, q_b, kv_b, gsem_b)

        @pl.when(chunk_of(t0) < NCHUNKS)
        def _():
            consume(t0, tt, idx_a, sidx_a, q_a, kv_a, upd_a, gsem_a, ssem_a)

        @pl.when(chunk_of(t0 + 2) < NCHUNKS)
        def _():
            issue(t0 + 2, idx_a, q_a, kv_a, gsem_a)

        @pl.when(chunk_of(t1) < NCHUNKS)
        def _():
            consume(t1, tt, idx_b, sidx_b, q_b, kv_b, upd_b, gsem_b, ssem_b)

        return carry

    lax.fori_loop(0, NT // 2, pair_body, 0)

    pltpu.make_async_copy(upd_a, acc.at[sidx_a.at[0]], ssem_a).wait()
    pltpu.make_async_copy(upd_b, acc.at[sidx_b.at[0]], ssem_b).wait()

    plsc.subcore_barrier()

    @pl.when(s == 0)
    def _():
        pltpu.sync_copy(acc, out_hbm.at[c])


@functools.cache
def _edge_kernel():
    mesh = plsc.VectorSubcoreMesh(core_axis_name="c", subcore_axis_name="s",
                                  num_cores=NC, num_subcores=NS)
    return pl.kernel(
        _edge_body,
        out_type=jax.ShapeDtypeStruct((NC, N, ACC_W), jnp.float32),
        mesh=mesh,
        compiler_params=pltpu.CompilerParams(needs_layout_passes=False,
                                             use_tc_tiling_on_sc=False),
        scratch_types=[
            pltpu.VMEM((2, K), jnp.int32),
            pltpu.VMEM((2, K), jnp.int32),
            pltpu.VMEM((1, K), jnp.int32),
            pltpu.VMEM((1, K), jnp.int32),
            pltpu.VMEM((K, QW), jnp.float32),
            pltpu.VMEM((K, QW), jnp.float32),
            pltpu.VMEM((K, KVW), jnp.float32),
            pltpu.VMEM((K, KVW), jnp.float32),
            pltpu.VMEM((K, ACC_W), jnp.float32),
            pltpu.VMEM((K, ACC_W), jnp.float32),
            pltpu.VMEM((HEADS_C, K), jnp.float32),
            pltpu.VMEM_SHARED((N, ACC_W), jnp.float32),
            pltpu.SemaphoreType.DMA,
            pltpu.SemaphoreType.DMA,
            pltpu.SemaphoreType.DMA,
            pltpu.SemaphoreType.DMA,
        ],
    )


_PB = 2000
_PNB = N // _PB


def _post1_body(accs_ref, x_ref, wo_ref, bo_ref, h_ref, stats_ref):
    rb = _PB
    num = jnp.concatenate([accs_ref[0, :, :QW], accs_ref[1, :, :QW]], axis=1)
    den8 = jnp.concatenate([accs_ref[0, :, QW:QW + HEADS_C],
                            accs_ref[1, :, QW:QW + HEADS_C]], axis=1)
    den = jnp.reshape(
        jnp.broadcast_to(den8[:, :, None], (rb, HEADS, HD)), (rb, HIDDEN))
    att = jnp.where(den != 0.0, num / den, 0.0)
    out = jnp.dot(att, wo_ref[...], preferred_element_type=jnp.float32,
                  precision=lax.Precision.HIGHEST) + bo_ref[...]
    h = out + x_ref[...]
    h_ref[...] = h
    part = jnp.concatenate([jnp.sum(h, axis=0, keepdims=True),
                            jnp.sum(h * h, axis=0, keepdims=True)], axis=0)

    @pl.when(pl.program_id(0) == 0)
    def _():
        stats_ref[...] = part

    @pl.when(pl.program_id(0) != 0)
    def _():
        stats_ref[...] = stats_ref[...] + part


def _post2_body(h_ref, stats_ref, gnw_ref, gnb_ref, gms_ref, out_ref):
    h = h_ref[...]
    gms = gms_ref[...]
    mean = stats_ref[0:1] * (1.0 / N)
    ex2 = stats_ref[1:2] * (1.0 / N)
    gm = gms * mean
    var = ex2 - 2.0 * gm * mean + gm * gm
    hn = gnw_ref[...] * (h - gm) / jnp.sqrt(var + EPS) + gnb_ref[...]
    out_ref[...] = 0.5 * hn * (1.0 + lax.erf(hn * (2.0 ** -0.5)))


def _tc_post(accs, x, Wo, bo, gnw, gnb, gms):
    h, stats = pl.pallas_call(
        _post1_body,
        grid=(_PNB,),
        in_specs=[
            pl.BlockSpec((NC, _PB, ACC_W), lambda j: (0, j, 0)),
            pl.BlockSpec((_PB, HIDDEN), lambda j: (j, 0)),
            pl.BlockSpec((HIDDEN, HIDDEN), lambda j: (0, 0)),
            pl.BlockSpec((1, HIDDEN), lambda j: (0, 0)),
        ],
        out_specs=[
            pl.BlockSpec((_PB, HIDDEN), lambda j: (j, 0)),
            pl.BlockSpec((2, HIDDEN), lambda j: (0, 0)),
        ],
        out_shape=[
            jax.ShapeDtypeStruct((N, HIDDEN), jnp.float32),
            jax.ShapeDtypeStruct((2, HIDDEN), jnp.float32),
        ],
    )(accs, x, Wo, bo.reshape(1, HIDDEN))
    return pl.pallas_call(
        _post2_body,
        grid=(_PNB,),
        in_specs=[
            pl.BlockSpec((_PB, HIDDEN), lambda j: (j, 0)),
            pl.BlockSpec((2, HIDDEN), lambda j: (0, 0)),
            pl.BlockSpec((1, HIDDEN), lambda j: (0, 0)),
            pl.BlockSpec((1, HIDDEN), lambda j: (0, 0)),
            pl.BlockSpec((1, HIDDEN), lambda j: (0, 0)),
        ],
        out_specs=pl.BlockSpec((_PB, HIDDEN), lambda j: (j, 0)),
        out_shape=jax.ShapeDtypeStruct((N, HIDDEN), jnp.float32),
    )(h, stats, gnw.reshape(1, HIDDEN), gnb.reshape(1, HIDDEN),
      gms.reshape(1, HIDDEN))


@jax.jit
def kernel(x, edge_index, WQ, WK, WV, Wo, bo, gn_weight, gn_bias,
           gn_mean_scale):
    Wq2 = jnp.stack([WQ[:, :QW], WQ[:, QW:]])
    Wkv2 = jnp.stack(
        [jnp.concatenate([WK[:, :QW], WV[:, :QW]], axis=1),
         jnp.concatenate([WK[:, QW:], WV[:, QW:]], axis=1)])
    q2, kv2 = _tc_pre(x, Wq2, Wkv2)
    edges2 = edge_index.reshape(2, NCHUNKS, K).transpose(1, 0, 2)
    zeros = jnp.zeros((N, ACC_W), jnp.float32)
    accs = _edge_kernel()(q2, kv2, edges2, zeros)
    return _tc_post(accs, x, Wo, bo, gn_weight, gn_bias, gn_mean_scale)


# speedup vs baseline: 1.8749x; 1.8749x over previous
import functools

import jax
import jax.numpy as jnp
from jax import lax
from jax.experimental import pallas as pl
from jax.experimental.pallas import tpu as pltpu
from jax.experimental.pallas import tpu_sc as plsc

N = 10000
E = 320000
HIDDEN = 128
HEADS = 8
HD = HIDDEN // HEADS
EPS = 1e-5

NC = 2
NS = 16
L = 16

HEADS_C = HEADS // NC
QW = HEADS_C * HD
KVW = 2 * QW
ACC_W = 80

K = 128
NCHUNKS = E // K
NT = 2 * (-(-(-(-NCHUNKS // NS)) // 2))


def _pre_body(x_ref, wq_ref, wkv_ref, q_ref, kv_ref):
    xb = x_ref[...]
    q_ref[0] = jnp.dot(xb, wq_ref[0], preferred_element_type=jnp.float32,
                       precision=lax.Precision.HIGHEST)
    kv_ref[0] = jnp.dot(xb, wkv_ref[0], preferred_element_type=jnp.float32,
                        precision=lax.Precision.HIGHEST)


def _tc_pre(x, Wq2, Wkv2):
    nb = 5
    rb = N // nb
    return pl.pallas_call(
        _pre_body,
        grid=(NC, nb),
        in_specs=[
            pl.BlockSpec((rb, HIDDEN), lambda c, j: (j, 0)),
            pl.BlockSpec((1, HIDDEN, QW), lambda c, j: (c, 0, 0)),
            pl.BlockSpec((1, HIDDEN, KVW), lambda c, j: (c, 0, 0)),
        ],
        out_specs=[
            pl.BlockSpec((1, rb, QW), lambda c, j: (c, j, 0)),
            pl.BlockSpec((1, rb, KVW), lambda c, j: (c, j, 0)),
        ],
        out_shape=[
            jax.ShapeDtypeStruct((NC, N, QW), jnp.float32),
            jax.ShapeDtypeStruct((NC, N, KVW), jnp.float32),
        ],
    )(x, Wq2, Wkv2)


def _compute_upd(est, q_rows, kv_rows, upd):
    lanes = lax.iota(jnp.int32, L)

    @plsc.parallel_loop(0, K // L, 1, unroll=2)
    def group_body(g):
        e0 = g * L
        for h in range(HEADS_C):
            sv = jnp.zeros((L,), jnp.float32)
            for i in range(L):
                e = e0 + i
                qv = q_rows[e, pl.ds(h * HD, HD)]
                kk = kv_rows[e, pl.ds(h * HD, HD)]
                sv = jnp.where(lanes == i, jnp.sum(qv * kk), sv)
            est[h, pl.ds(e0, L)] = jnp.exp(jnp.minimum(sv * 0.25, 60.0))
        esv = [est[h, pl.ds(e0, L)] for h in range(HEADS_C)]
        for i in range(L):
            e = e0 + i
            den = jnp.zeros((L,), jnp.float32)
            for h in range(HEADS_C):
                sb = esv[h].at[jnp.full((L,), i, jnp.int32)].get(
                    mode="promise_in_bounds")
                vv = kv_rows[e, pl.ds(QW + h * HD, HD)]
                upd[e, pl.ds(h * HD, HD)] = vv * sb
                den = jnp.where(lanes == h, sb, den)
            upd[e, pl.ds(QW, L)] = den


def _edge_body(q_hbm, kv_hbm, edges2_hbm, zeros_hbm, out_hbm,
               idx_a, idx_b, sidx_a, sidx_b, q_a, q_b, kv_a, kv_b,
               upd_a, upd_b, est, acc, gsem_a, gsem_b, ssem_a, ssem_b):
    c = lax.axis_index("c")
    s = lax.axis_index("s")

    @pl.when(s == 0)
    def _():
        pltpu.sync_copy(zeros_hbm, acc)

    plsc.subcore_barrier()

    def chunk_of(t):
        return s + t * NS

    def issue(t, idx2, qb, kvb, gsem):
        pltpu.sync_copy(edges2_hbm.at[chunk_of(t)], idx2)
        pltpu.async_copy(kv_hbm.at[c].at[idx2.at[0]], kvb, gsem)
        pltpu.async_copy(q_hbm.at[c].at[idx2.at[1]], qb, gsem)

    def consume(t, tt, idx2, sidx, qb, kvb, upd, gsem, ssem):
        pltpu.make_async_copy(kv_hbm.at[c].at[idx2.at[0]], kvb, gsem).wait()
        pltpu.make_async_copy(q_hbm.at[c].at[idx2.at[1]], qb, gsem).wait()

        @pl.when(tt > 0)
        def _():
            pltpu.make_async_copy(upd, acc.at[sidx.at[0]], ssem).wait()

        for i in range(K // L):
            sidx[0, pl.ds(i * L, L)] = idx2[1, pl.ds(i * L, L)]
        _compute_upd(est, qb, kvb, upd)
        pltpu.async_copy(upd, acc.at[sidx.at[0]], ssem, add=True)

    @pl.when(chunk_of(0) < NCHUNKS)
    def _():
        issue(0, idx_a, q_a, kv_a, gsem_a)

    def pair_body(tt, carry):
        t0 = 2 * tt
        t1 = 2 * tt + 1

        @pl.when(chunk_of(t1) < NCHUNKS)
        def _():
            issue(t1, idx_b, q_b, kv_b, gsem_b)

        @pl.when(chunk_of(t0) < NCHUNKS)
        def _():
            consume(t0, tt, idx_a, sidx_a, q_a, kv_a, upd_a, gsem_a, ssem_a)

        @pl.when(chunk_of(t0 + 2) < NCHUNKS)
        def _():
            issue(t0 + 2, idx_a, q_a, kv_a, gsem_a)

        @pl.when(chunk_of(t1) < NCHUNKS)
        def _():
            consume(t1, tt, idx_b, sidx_b, q_b, kv_b, upd_b, gsem_b, ssem_b)

        return carry

    lax.fori_loop(0, NT // 2, pair_body, 0)

    pltpu.make_async_copy(upd_a, acc.at[sidx_a.at[0]], ssem_a).wait()
    pltpu.make_async_copy(upd_b, acc.at[sidx_b.at[0]], ssem_b).wait()

    plsc.subcore_barrier()

    @pl.when(s == 0)
    def _():
        pltpu.sync_copy(acc, out_hbm.at[c])


@functools.cache
def _edge_kernel():
    mesh = plsc.VectorSubcoreMesh(core_axis_name="c", subcore_axis_name="s",
                                  num_cores=NC, num_subcores=NS)
    return pl.kernel(
        _edge_body,
        out_type=jax.ShapeDtypeStruct((NC, N, ACC_W), jnp.float32),
        mesh=mesh,
        compiler_params=pltpu.CompilerParams(needs_layout_passes=False,
                                             use_tc_tiling_on_sc=False),
        scratch_types=[
            pltpu.VMEM((2, K), jnp.int32),
            pltpu.VMEM((2, K), jnp.int32),
            pltpu.VMEM((1, K), jnp.int32),
            pltpu.VMEM((1, K), jnp.int32),
            pltpu.VMEM((K, QW), jnp.float32),
            pltpu.VMEM((K, QW), jnp.float32),
            pltpu.VMEM((K, KVW), jnp.float32),
            pltpu.VMEM((K, KVW), jnp.float32),
            pltpu.VMEM((K, ACC_W), jnp.float32),
            pltpu.VMEM((K, ACC_W), jnp.float32),
            pltpu.VMEM((HEADS_C, K), jnp.float32),
            pltpu.VMEM_SHARED((N, ACC_W), jnp.float32),
            pltpu.SemaphoreType.DMA,
            pltpu.SemaphoreType.DMA,
            pltpu.SemaphoreType.DMA,
            pltpu.SemaphoreType.DMA,
        ],
    )


_PB = 2000
_PNB = N // _PB


def _post1_body(accs_ref, x_ref, wo_ref, bo_ref, h_ref, stats_ref):
    rb = _PB
    num = jnp.concatenate([accs_ref[0, :, :QW], accs_ref[1, :, :QW]], axis=1)
    den8 = jnp.concatenate([accs_ref[0, :, QW:QW + HEADS_C],
                            accs_ref[1, :, QW:QW + HEADS_C]], axis=1)
    den = jnp.reshape(
        jnp.broadcast_to(den8[:, :, None], (rb, HEADS, HD)), (rb, HIDDEN))
    att = jnp.where(den != 0.0, num / den, 0.0)
    out = jnp.dot(att, wo_ref[...], preferred_element_type=jnp.float32,
                  precision=lax.Precision.HIGHEST) + bo_ref[...]
    h = out + x_ref[...]
    h_ref[...] = h
    part = jnp.concatenate([jnp.sum(h, axis=0, keepdims=True),
                            jnp.sum(h * h, axis=0, keepdims=True)], axis=0)

    @pl.when(pl.program_id(0) == 0)
    def _():
        stats_ref[...] = part

    @pl.when(pl.program_id(0) != 0)
    def _():
        stats_ref[...] = stats_ref[...] + part


def _post2_body(h_ref, stats_ref, gnw_ref, gnb_ref, gms_ref, out_ref):
    h = h_ref[...]
    gms = gms_ref[...]
    mean = stats_ref[0:1] * (1.0 / N)
    ex2 = stats_ref[1:2] * (1.0 / N)
    gm = gms * mean
    var = ex2 - 2.0 * gm * mean + gm * gm
    hn = gnw_ref[...] * (h - gm) / jnp.sqrt(var + EPS) + gnb_ref[...]
    out_ref[...] = 0.5 * hn * (1.0 + lax.erf(hn * (2.0 ** -0.5)))


def _tc_post(accs, x, Wo, bo, gnw, gnb, gms):
    h, stats = pl.pallas_call(
        _post1_body,
        grid=(_PNB,),
        in_specs=[
            pl.BlockSpec((NC, _PB, ACC_W), lambda j: (0, j, 0)),
            pl.BlockSpec((_PB, HIDDEN), lambda j: (j, 0)),
            pl.BlockSpec((HIDDEN, HIDDEN), lambda j: (0, 0)),
            pl.BlockSpec((1, HIDDEN), lambda j: (0, 0)),
        ],
        out_specs=[
            pl.BlockSpec((_PB, HIDDEN), lambda j: (j, 0)),
            pl.BlockSpec((2, HIDDEN), lambda j: (0, 0)),
        ],
        out_shape=[
            jax.ShapeDtypeStruct((N, HIDDEN), jnp.float32),
            jax.ShapeDtypeStruct((2, HIDDEN), jnp.float32),
        ],
    )(accs, x, Wo, bo.reshape(1, HIDDEN))
    return pl.pallas_call(
        _post2_body,
        grid=(_PNB,),
        in_specs=[
            pl.BlockSpec((_PB, HIDDEN), lambda j: (j, 0)),
            pl.BlockSpec((2, HIDDEN), lambda j: (0, 0)),
            pl.BlockSpec((1, HIDDEN), lambda j: (0, 0)),
            pl.BlockSpec((1, HIDDEN), lambda j: (0, 0)),
            pl.BlockSpec((1, HIDDEN), lambda j: (0, 0)),
        ],
        out_specs=pl.BlockSpec((_PB, HIDDEN), lambda j: (j, 0)),
        out_shape=jax.ShapeDtypeStruct((N, HIDDEN), jnp.float32),
    )(h, stats, gnw.reshape(1, HIDDEN), gnb.reshape(1, HIDDEN),
      gms.reshape(1, HIDDEN))


@jax.jit
def kernel(x, edge_index, WQ, WK, WV, Wo, bo, gn_weight, gn_bias,
           gn_mean_scale):
    Wq2 = jnp.stack([WQ[:, :QW], WQ[:, QW:]])
    Wkv2 = jnp.stack(
        [jnp.concatenate([WK[:, :QW], WV[:, :QW]], axis=1),
         jnp.concatenate([WK[:, QW:], WV[:, QW:]], axis=1)])
    q2, kv2 = _tc_pre(x, Wq2, Wkv2)
    edges2 = edge_index.reshape(2, NCHUNKS, K).transpose(1, 0, 2)
    zeros = jnp.zeros((N, ACC_W), jnp.float32)
    accs = _edge_kernel()(q2, kv2, edges2, zeros)
    return _tc_post(accs, x, Wo, bo, gn_weight, gn_bias, gn_mean_scale)
